# Initial kernel scaffold; baseline (speedup 1.0000x reference)
import jax
import jax.numpy as jnp
from jax.experimental import pallas as pl


def kernel(x_transaction, x_account, params, edge_index_t2a, edge_index_a2t):
    raise NotImplementedError("write your pallas kernel here")


# speedup vs baseline: 1.0350x; 1.0350x over previous
import functools
import jax
import jax.numpy as jnp
from jax.experimental import pallas as pl
from jax.experimental.pallas import tpu as pltpu

H = 4
D = 16
HID = 64
N_NODES = 50000
BN = 1000


def _proj_body(x_ref, w_ref, b_ref, out_ref):
    out_ref[...] = (
        jnp.dot(x_ref[...], w_ref[...], preferred_element_type=jnp.float32)
        + b_ref[...]
    )


def _proj(x, w, b):
    n, p = x.shape[0], w.shape[1]
    grid = n // BN
    return pl.pallas_call(
        _proj_body,
        grid=(grid,),
        in_specs=[
            pl.BlockSpec((BN, x.shape[1]), lambda i: (i, 0)),
            pl.BlockSpec((w.shape[0], p), lambda i: (0, 0)),
            pl.BlockSpec((1, p), lambda i: (0, 0)),
        ],
        out_specs=pl.BlockSpec((BN, p), lambda i: (i, 0)),
        out_shape=jax.ShapeDtypeStruct((n, p), jnp.float32),
    )(x, w, b)


def _out_stage_body(acc_ref, x_ref, wa_ref, ba_ref, g_ref, out_ref, *, relu,
                    wcls=False):
    parts = []
    for h in range(H):
        num = acc_ref[:, h * D:(h + 1) * D]
        den = acc_ref[:, 64 + h:65 + h]
        parts.append(num / (den + 1e-16))
    o = jnp.concatenate(parts, axis=1)
    o = jax.nn.gelu(o)
    o = jnp.dot(o, wa_ref[...], preferred_element_type=jnp.float32) + ba_ref[...]
    g = g_ref[0, 0]
    res = g * o + (1.0 - g) * x_ref[...]
    if relu:
        res = jnp.maximum(res, 0.0)
    out_ref[...] = res


def _out_stage(acc, x, wa, ba, g, relu):
    n = x.shape[0]
    grid = n // BN
    return pl.pallas_call(
        functools.partial(_out_stage_body, relu=relu),
        grid=(grid,),
        in_specs=[
            pl.BlockSpec((BN, 80), lambda i: (i, 0)),
            pl.BlockSpec((BN, HID), lambda i: (i, 0)),
            pl.BlockSpec((HID, HID), lambda i: (0, 0)),
            pl.BlockSpec((1, HID), lambda i: (0, 0)),
            pl.BlockSpec((1, 1), lambda i: (0, 0)),
        ],
        out_specs=pl.BlockSpec((BN, HID), lambda i: (i, 0)),
        out_shape=jax.ShapeDtypeStruct((n, HID), jnp.float32),
    )(acc, x, wa, ba, g)


def _out_cls_body(acc_ref, x_ref, wa_ref, ba_ref, g_ref, wc_ref, bc_ref,
                  out_ref):
    parts = []
    for h in range(H):
        num = acc_ref[:, h * D:(h + 1) * D]
        den = acc_ref[:, 64 + h:65 + h]
        parts.append(num / (den + 1e-16))
    o = jnp.concatenate(parts, axis=1)
    o = jax.nn.gelu(o)
    o = jnp.dot(o, wa_ref[...], preferred_element_type=jnp.float32) + ba_ref[...]
    g = g_ref[0, 0]
    res = g * o + (1.0 - g) * x_ref[...]
    out_ref[...] = (
        jnp.dot(res, wc_ref[...], preferred_element_type=jnp.float32)
        + bc_ref[...]
    )


def _out_cls(acc, x, wa, ba, g, wc, bc):
    n = x.shape[0]
    grid = n // BN
    return pl.pallas_call(
        _out_cls_body,
        grid=(grid,),
        in_specs=[
            pl.BlockSpec((BN, 80), lambda i: (i, 0)),
            pl.BlockSpec((BN, HID), lambda i: (i, 0)),
            pl.BlockSpec((HID, HID), lambda i: (0, 0)),
            pl.BlockSpec((1, HID), lambda i: (0, 0)),
            pl.BlockSpec((1, 1), lambda i: (0, 0)),
            pl.BlockSpec((HID, 128), lambda i: (0, 0)),
            pl.BlockSpec((1, 128), lambda i: (0, 0)),
        ],
        out_specs=pl.BlockSpec((BN, 128), lambda i: (i, 0)),
        out_shape=jax.ShapeDtypeStruct((n, 128), jnp.float32),
    )(acc, x, wa, ba, g, wc, bc)


def _edge_pass(q_dst, kv_src, row, col, n_dst):
    k = kv_src[:, :64].reshape(-1, H, D)
    v = kv_src[:, 64:].reshape(-1, H, D)
    qe = q_dst.reshape(-1, H, D)[col]
    ke = k[row]
    ve = v[row]
    ex = jnp.exp((qe * ke).sum(-1))
    num = jax.ops.segment_sum(ve * ex[:, :, None], col, num_segments=n_dst)
    den = jax.ops.segment_sum(ex, col, num_segments=n_dst)
    acc = jnp.concatenate(
        [num.reshape(-1, HID), den, jnp.zeros((n_dst, 12), jnp.float32)], axis=1)
    return acc


def _fold_weights(pn, pe_out, pe_in):
    wq = pn["Wq"]
    bq = pn["bq"]
    scale = pe_out["p_rel"] / jnp.sqrt(float(D))
    wk = jnp.einsum("ihd,hde->ihe", pn["Wk"].reshape(-1, H, D), pe_out["a_rel"])
    wk = (wk * scale[None, :, None]).reshape(-1, HID)
    bk = jnp.einsum("hd,hde->he", pn["bk"].reshape(H, D), pe_out["a_rel"])
    bk = (bk * scale[:, None]).reshape(HID)
    wv = jnp.einsum("ihd,hde->ihe", pn["Wv"].reshape(-1, H, D), pe_out["m_rel"])
    wv = wv.reshape(-1, HID)
    bv = jnp.einsum("hd,hde->he", pn["bv"].reshape(H, D), pe_out["m_rel"])
    bv = bv.reshape(HID)
    w = jnp.concatenate([wq, wk, wv], axis=1)
    b = jnp.concatenate([bq, bk, bv])[None, :]
    return w, b


def kernel(x_transaction, x_account, params, edge_index_t2a, edge_index_a2t):
    p1, p2 = params["conv1"], params["conv2"]
    e_t2a = "transaction__to__account"
    e_a2t = "account__to__transaction"

    row_t2a, col_t2a = edge_index_t2a[0], edge_index_t2a[1]
    row_a2t, col_a2t = edge_index_a2t[0], edge_index_a2t[1]

    wt, bt = _fold_weights(p1["nodes"]["transaction"], p1["edges"][e_t2a], None)
    wa_, ba_ = _fold_weights(p1["nodes"]["account"], p1["edges"][e_a2t], None)
    pt = _proj(x_transaction, wt, bt)
    pa = _proj(x_account, wa_, ba_)

    acc_a = _edge_pass(pa[:, :64], pt[:, 64:], row_t2a, col_t2a, N_NODES)
    acc_t = _edge_pass(pt[:, :64], pa[:, 64:], row_a2t, col_a2t, N_NODES)

    pn = p1["nodes"]["transaction"]
    h_t = _out_stage(acc_t, x_transaction, pn["Wa"], pn["ba"][None, :],
                     jax.nn.sigmoid(pn["skip"]).reshape(1, 1), relu=True)
    pn = p1["nodes"]["account"]
    h_a = _out_stage(acc_a, x_account, pn["Wa"], pn["ba"][None, :],
                     jax.nn.sigmoid(pn["skip"]).reshape(1, 1), relu=True)

    wt2, bt2 = _fold_weights(p2["nodes"]["transaction"], p2["edges"][e_t2a], None)
    wa2, ba2 = _fold_weights(p2["nodes"]["account"], p2["edges"][e_a2t], None)
    pt2 = _proj(h_t, wt2[:, :64], bt2[:, :64])
    pa2 = _proj(h_a, wa2[:, 64:], ba2[:, 64:])

    acc_t2 = _edge_pass(pt2, pa2, row_a2t, col_a2t, N_NODES)

    pn = p2["nodes"]["transaction"]
    wc = jnp.zeros((HID, 128), jnp.float32).at[:, :1].set(params["cls_W"])
    bc = jnp.zeros((1, 128), jnp.float32).at[0, 0].set(params["cls_b"][0])
    out = _out_cls(acc_t2, h_t, pn["Wa"], pn["ba"][None, :],
                   jax.nn.sigmoid(pn["skip"]).reshape(1, 1), wc, bc)
    return out[:, :1]


# speedup vs baseline: 21.6006x; 20.8707x over previous
import functools
import jax
import jax.numpy as jnp
from jax import lax
from jax.experimental import pallas as pl
from jax.experimental.pallas import tpu as pltpu
from jax.experimental.pallas import tpu_sc as plsc

H = 4
D = 16
HID = 64
N_NODES = 50000
BN = 1000

E_TOTAL = 800000
SC_HALF = 25088
NPAD = 2 * SC_HALF
SUBC_ROWS = SC_HALF // 16
EC = 80
EPS = E_TOTAL // 16
NCH = EPS // EC


def _proj_body(x_ref, w_ref, b_ref, q_ref, k_ref, v_ref):
    o = (jnp.dot(x_ref[...], w_ref[...], preferred_element_type=jnp.float32)
         + b_ref[...])
    q_ref[...] = o[:, :HID]
    k_ref[...] = o[:, HID:2 * HID]
    v_ref[...] = o[:, 2 * HID:]


def _proj(x, w, b):
    n = x.shape[0]
    grid = n // BN
    sds = jax.ShapeDtypeStruct((n, HID), jnp.float32)
    return pl.pallas_call(
        _proj_body,
        grid=(grid,),
        in_specs=[
            pl.BlockSpec((BN, HID), lambda i: (i, 0)),
            pl.BlockSpec((HID, 3 * HID), lambda i: (0, 0)),
            pl.BlockSpec((1, 3 * HID), lambda i: (0, 0)),
        ],
        out_specs=[pl.BlockSpec((BN, HID), lambda i: (i, 0))] * 3,
        out_shape=[sds, sds, sds],
    )(x, w, b)


def _softmax_out(num_ref, den_ref):
    parts = []
    for h in range(H):
        num = num_ref[:, h * D:(h + 1) * D]
        den = den_ref[:, h:h + 1]
        parts.append(num / (den + 1e-16))
    return jnp.concatenate(parts, axis=1)


def _out_stage_body(num_ref, den_ref, x_ref, wa_ref, ba_ref, g_ref, out_ref,
                    *, relu):
    o = jax.nn.gelu(_softmax_out(num_ref, den_ref))
    o = jnp.dot(o, wa_ref[...], preferred_element_type=jnp.float32) + ba_ref[...]
    g = g_ref[0, 0]
    res = g * o + (1.0 - g) * x_ref[...]
    if relu:
        res = jnp.maximum(res, 0.0)
    out_ref[...] = res


def _out_stage(num, den, x, wa, ba, g, relu):
    n = x.shape[0]
    grid = n // BN
    return pl.pallas_call(
        functools.partial(_out_stage_body, relu=relu),
        grid=(grid,),
        in_specs=[
            pl.BlockSpec((BN, HID), lambda i: (i, 0)),
            pl.BlockSpec((BN, 16), lambda i: (i, 0)),
            pl.BlockSpec((BN, HID), lambda i: (i, 0)),
            pl.BlockSpec((HID, HID), lambda i: (0, 0)),
            pl.BlockSpec((1, HID), lambda i: (0, 0)),
            pl.BlockSpec((1, 1), lambda i: (0, 0)),
        ],
        out_specs=pl.BlockSpec((BN, HID), lambda i: (i, 0)),
        out_shape=jax.ShapeDtypeStruct((n, HID), jnp.float32),
    )(num, den, x, wa, ba, g)


def _out_cls_body(num_ref, den_ref, x_ref, wa_ref, ba_ref, g_ref, wc_ref,
                  bc_ref, out_ref):
    o = jax.nn.gelu(_softmax_out(num_ref, den_ref))
    o = jnp.dot(o, wa_ref[...], preferred_element_type=jnp.float32) + ba_ref[...]
    g = g_ref[0, 0]
    res = g * o + (1.0 - g) * x_ref[...]
    out_ref[...] = (
        jnp.dot(res, wc_ref[...], preferred_element_type=jnp.float32)
        + bc_ref[...]
    )


def _out_cls(num, den, x, wa, ba, g, wc, bc):
    n = x.shape[0]
    grid = n // BN
    return pl.pallas_call(
        _out_cls_body,
        grid=(grid,),
        in_specs=[
            pl.BlockSpec((BN, HID), lambda i: (i, 0)),
            pl.BlockSpec((BN, 16), lambda i: (i, 0)),
            pl.BlockSpec((BN, HID), lambda i: (i, 0)),
            pl.BlockSpec((HID, HID), lambda i: (0, 0)),
            pl.BlockSpec((1, HID), lambda i: (0, 0)),
            pl.BlockSpec((1, 1), lambda i: (0, 0)),
            pl.BlockSpec((HID, 128), lambda i: (0, 0)),
            pl.BlockSpec((1, 128), lambda i: (0, 0)),
        ],
        out_specs=pl.BlockSpec((BN, 128), lambda i: (i, 0)),
        out_shape=jax.ShapeDtypeStruct((n, 128), jnp.float32),
    )(num, den, x, wa, ba, g, wc, bc)


_SC_MESH = plsc.VectorSubcoreMesh(core_axis_name="c", subcore_axis_name="s")
_SC_PARAMS = pltpu.CompilerParams(use_tc_tiling_on_sc=False)


def _shuf(x, idx):
    return lax.gather(
        x, idx[:, None],
        dimension_numbers=lax.GatherDimensionNumbers(
            offset_dims=(), collapsed_slice_dims=(0,), start_index_map=(0,)),
        slice_sizes=(1,), mode=lax.GatherScatterMode.PROMISE_IN_BOUNDS)


def _zero_shared(msg, shared, s, width):
    zero16 = jnp.zeros((16,), jnp.float32)

    def _zrow(i, carry):
        for j in range(width // 16):
            msg[i, pl.ds(j * 16, 16)] = zero16
        return carry
    lax.fori_loop(0, EC, _zrow, 0)
    zb = s * SUBC_ROWS
    for j in range(SUBC_ROWS // EC):
        pltpu.sync_copy(msg, shared.at[pl.ds(zb + j * EC, EC)])
    rem = SUBC_ROWS % EC
    if rem:
        pltpu.sync_copy(msg.at[pl.ds(0, rem)],
                        shared.at[pl.ds(zb + SUBC_ROWS - rem, rem)])

    @pl.when(s == 0)
    def _():
        pltpu.sync_copy(msg.at[pl.ds(0, 8)], shared.at[pl.ds(SC_HALF, 8)])


def _copy_out(msg, shared, out_hbm, s, base_node):
    r0 = s * SUBC_ROWS
    for j in range(SUBC_ROWS // EC):
        pltpu.sync_copy(shared.at[pl.ds(r0 + j * EC, EC)], msg)
        pltpu.sync_copy(msg, out_hbm.at[pl.ds(base_node + r0 + j * EC, EC)])
    rem = SUBC_ROWS % EC
    if rem:
        off = r0 + SUBC_ROWS - rem
        pltpu.sync_copy(shared.at[pl.ds(off, rem)], msg.at[pl.ds(0, rem)])
        pltpu.sync_copy(msg.at[pl.ds(0, rem)],
                        out_hbm.at[pl.ds(base_node + off, rem)])


def _localize(colb, c):
    base = c * SC_HALF
    for j in range(EC // 16):
        cv = colb[pl.ds(j * 16, 16)]
        loc = cv - base
        ok = (loc >= 0) & (loc < SC_HALF)
        colb[pl.ds(j * 16, 16)] = jnp.where(ok, loc, SC_HALF)


@functools.partial(
    pl.kernel,
    out_type=(jax.ShapeDtypeStruct((E_TOTAL, 16), jnp.float32),
              jax.ShapeDtypeStruct((NPAD, 16), jnp.float32)),
    mesh=_SC_MESH, compiler_params=_SC_PARAMS,
    scratch_types=[pltpu.VMEM((EC,), jnp.int32),
                   pltpu.VMEM((EC,), jnp.int32),
                   pltpu.VMEM((EC, 64), jnp.float32),
                   pltpu.VMEM((EC, 64), jnp.float32),
                   pltpu.VMEM((EC, 16), jnp.float32),
                   pltpu.VMEM_SHARED((SC_HALF + 8, 16), jnp.float32),
                   pltpu.SemaphoreType.DMA])
def _edge_kernel_a(q_hbm, k_hbm, row_hbm, col_hbm, ex_hbm, den_hbm,
                   rowb, colb, qb, kb, exm, shared, sem):
    c = lax.axis_index("c")
    s = lax.axis_index("s")
    lanes = lax.iota(jnp.int32, 16)
    zero16 = jnp.zeros((16,), jnp.float32)
    _zero_shared(exm, shared, s, 16)
    plsc.subcore_barrier()

    def _chunk(t, carry):
        eb = s * EPS + t * EC
        pltpu.sync_copy(row_hbm.at[pl.ds(eb, EC)], rowb)
        pltpu.sync_copy(col_hbm.at[pl.ds(eb, EC)], colb)
        cp1 = pltpu.async_copy(k_hbm.at[rowb], kb, sem)
        cp2 = pltpu.async_copy(q_hbm.at[colb], qb, sem)
        cp1.wait()
        cp2.wait()

        def _edge(e, carry2):
            den = zero16
            for h in range(H):
                prod = qb[e, pl.ds(h * 16, 16)] * kb[e, pl.ds(h * 16, 16)]
                for shift in (8, 4, 2, 1):
                    prod = prod + _shuf(prod, lanes ^ shift)
                ex = jnp.exp(prod)
                den = den + jnp.where(lanes == h, ex, 0.0)
            exm[e, pl.ds(0, 16)] = den
            return carry2
        lax.fori_loop(0, EC, _edge, 0)

        @pl.when(c == 0)
        def _():
            pltpu.sync_copy(exm, ex_hbm.at[pl.ds(eb, EC)])
        _localize(colb, c)
        pltpu.sync_copy(exm, shared.at[colb], add=True)
        return carry
    lax.fori_loop(0, NCH, _chunk, 0)
    plsc.subcore_barrier()
    _copy_out(exm, shared, den_hbm, s, c * SC_HALF)


@functools.partial(
    pl.kernel,
    out_type=jax.ShapeDtypeStruct((NPAD, 64), jnp.float32),
    mesh=_SC_MESH, compiler_params=_SC_PARAMS,
    scratch_types=[pltpu.VMEM((EC,), jnp.int32),
                   pltpu.VMEM((EC,), jnp.int32),
                   pltpu.VMEM((EC, 64), jnp.float32),
                   pltpu.VMEM((EC, 16), jnp.float32),
                   pltpu.VMEM((EC, 64), jnp.float32),
                   pltpu.VMEM_SHARED((SC_HALF + 8, 64), jnp.float32),
                   pltpu.SemaphoreType.DMA])
def _edge_kernel_b(v_hbm, ex_hbm, row_hbm, col_hbm, num_hbm,
                   rowb, colb, vb, exb, msg, shared, sem):
    c = lax.axis_index("c")
    s = lax.axis_index("s")
    _zero_shared(msg, shared, s, 64)
    plsc.subcore_barrier()

    def _chunk(t, carry):
        eb = s * EPS + t * EC
        pltpu.sync_copy(row_hbm.at[pl.ds(eb, EC)], rowb)
        pltpu.sync_copy(col_hbm.at[pl.ds(eb, EC)], colb)
        pltpu.sync_copy(ex_hbm.at[pl.ds(eb, EC)], exb)
        pltpu.async_copy(v_hbm.at[rowb], vb, sem).wait()

        def _edge(e, carry2):
            exv = exb[e, pl.ds(0, 16)]
            for h in range(H):
                bex = _shuf(exv, jnp.full((16,), h, jnp.int32))
                msg[e, pl.ds(h * 16, 16)] = vb[e, pl.ds(h * 16, 16)] * bex
            return carry2
        lax.fori_loop(0, EC, _edge, 0)

        _localize(colb, c)
        pltpu.sync_copy(msg, shared.at[colb], add=True)
        return carry
    lax.fori_loop(0, NCH, _chunk, 0)
    plsc.subcore_barrier()
    _copy_out(msg, shared, num_hbm, s, c * SC_HALF)


def _edge_pass(q_dst, k_src, v_src, row, col):
    ex, den = _edge_kernel_a(q_dst, k_src, row, col)
    num = _edge_kernel_b(v_src, ex, row, col)
    return num, den


def _fold_weights(pn, pe_out):
    wq = pn["Wq"]
    bq = pn["bq"]
    scale = pe_out["p_rel"] / jnp.sqrt(float(D))
    wk = jnp.einsum("ihd,hde->ihe", pn["Wk"].reshape(-1, H, D), pe_out["a_rel"])
    wk = (wk * scale[None, :, None]).reshape(-1, HID)
    bk = jnp.einsum("hd,hde->he", pn["bk"].reshape(H, D), pe_out["a_rel"])
    bk = (bk * scale[:, None]).reshape(HID)
    wv = jnp.einsum("ihd,hde->ihe", pn["Wv"].reshape(-1, H, D), pe_out["m_rel"])
    wv = wv.reshape(-1, HID)
    bv = jnp.einsum("hd,hde->he", pn["bv"].reshape(H, D), pe_out["m_rel"])
    bv = bv.reshape(HID)
    w = jnp.concatenate([wq, wk, wv], axis=1)
    b = jnp.concatenate([bq, bk, bv])[None, :]
    return w, b


def kernel(x_transaction, x_account, params, edge_index_t2a, edge_index_a2t):
    p1, p2 = params["conv1"], params["conv2"]
    e_t2a = "transaction__to__account"
    e_a2t = "account__to__transaction"

    row_t2a, col_t2a = edge_index_t2a[0], edge_index_t2a[1]
    row_a2t, col_a2t = edge_index_a2t[0], edge_index_a2t[1]

    wt, bt = _fold_weights(p1["nodes"]["transaction"], p1["edges"][e_t2a])
    wa_, ba_ = _fold_weights(p1["nodes"]["account"], p1["edges"][e_a2t])
    qt, kt, vt = _proj(x_transaction, wt, bt)
    qa, ka, va = _proj(x_account, wa_, ba_)

    num_a, den_a = _edge_pass(qa, kt, vt, row_t2a, col_t2a)
    num_t, den_t = _edge_pass(qt, ka, va, row_a2t, col_a2t)

    pn = p1["nodes"]["transaction"]
    h_t = _out_stage(num_t, den_t, x_transaction, pn["Wa"], pn["ba"][None, :],
                     jax.nn.sigmoid(pn["skip"]).reshape(1, 1), relu=True)
    pn = p1["nodes"]["account"]
    h_a = _out_stage(num_a, den_a, x_account, pn["Wa"], pn["ba"][None, :],
                     jax.nn.sigmoid(pn["skip"]).reshape(1, 1), relu=True)

    wt2, bt2 = _fold_weights(p2["nodes"]["transaction"], p2["edges"][e_t2a])
    wa2, ba2 = _fold_weights(p2["nodes"]["account"], p2["edges"][e_a2t])
    qt2, _, _ = _proj(h_t, wt2, bt2)
    _, ka2, va2 = _proj(h_a, wa2, ba2)

    num_t2, den_t2 = _edge_pass(qt2, ka2, va2, row_a2t, col_a2t)

    pn = p2["nodes"]["transaction"]
    wc = jnp.zeros((HID, 128), jnp.float32).at[:, :1].set(params["cls_W"])
    bc = jnp.zeros((1, 128), jnp.float32).at[0, 0].set(params["cls_b"][0])
    out = _out_cls(num_t2, den_t2, h_t, pn["Wa"], pn["ba"][None, :],
                   jax.nn.sigmoid(pn["skip"]).reshape(1, 1), wc, bc)
    return out[:, :1]


# speedup vs baseline: 25.8113x; 1.1949x over previous
import functools
import jax
import jax.numpy as jnp
from jax import lax
from jax.experimental import pallas as pl
from jax.experimental.pallas import tpu as pltpu
from jax.experimental.pallas import tpu_sc as plsc

H = 4
D = 16
HID = 64
N_NODES = 50000
BN = 1000

E_TOTAL = 800000
SC_HALF = 25088
NPAD = 2 * SC_HALF
SUBC_ROWS = SC_HALF // 16
EC = 128
NCHT = E_TOTAL // EC


def _proj_body(x_ref, w_ref, b_ref, q_ref, k_ref, v_ref):
    o = (jnp.dot(x_ref[...], w_ref[...], preferred_element_type=jnp.float32)
         + b_ref[...])
    q_ref[...] = o[:, :HID]
    k_ref[...] = o[:, HID:2 * HID]
    v_ref[...] = o[:, 2 * HID:]


def _proj(x, w, b):
    n = x.shape[0]
    grid = n // BN
    sds = jax.ShapeDtypeStruct((n, HID), jnp.float32)
    return pl.pallas_call(
        _proj_body,
        grid=(grid,),
        in_specs=[
            pl.BlockSpec((BN, HID), lambda i: (i, 0)),
            pl.BlockSpec((HID, 3 * HID), lambda i: (0, 0)),
            pl.BlockSpec((1, 3 * HID), lambda i: (0, 0)),
        ],
        out_specs=[pl.BlockSpec((BN, HID), lambda i: (i, 0))] * 3,
        out_shape=[sds, sds, sds],
    )(x, w, b)


def _softmax_out(num_ref, den_ref):
    parts = []
    for h in range(H):
        num = num_ref[:, h * D:(h + 1) * D]
        den = den_ref[:, h:h + 1]
        parts.append(num / (den + 1e-16))
    return jnp.concatenate(parts, axis=1)


def _out_stage_body(num_ref, den_ref, x_ref, wa_ref, ba_ref, g_ref, out_ref,
                    *, relu):
    o = jax.nn.gelu(_softmax_out(num_ref, den_ref))
    o = jnp.dot(o, wa_ref[...], preferred_element_type=jnp.float32) + ba_ref[...]
    g = g_ref[0, 0]
    res = g * o + (1.0 - g) * x_ref[...]
    if relu:
        res = jnp.maximum(res, 0.0)
    out_ref[...] = res


def _out_stage(num, den, x, wa, ba, g, relu):
    n = x.shape[0]
    grid = n // BN
    return pl.pallas_call(
        functools.partial(_out_stage_body, relu=relu),
        grid=(grid,),
        in_specs=[
            pl.BlockSpec((BN, HID), lambda i: (i, 0)),
            pl.BlockSpec((BN, 16), lambda i: (i, 0)),
            pl.BlockSpec((BN, HID), lambda i: (i, 0)),
            pl.BlockSpec((HID, HID), lambda i: (0, 0)),
            pl.BlockSpec((1, HID), lambda i: (0, 0)),
            pl.BlockSpec((1, 1), lambda i: (0, 0)),
        ],
        out_specs=pl.BlockSpec((BN, HID), lambda i: (i, 0)),
        out_shape=jax.ShapeDtypeStruct((n, HID), jnp.float32),
    )(num, den, x, wa, ba, g)


def _out_cls_body(num_ref, den_ref, x_ref, wa_ref, ba_ref, g_ref, wc_ref,
                  bc_ref, out_ref):
    o = jax.nn.gelu(_softmax_out(num_ref, den_ref))
    o = jnp.dot(o, wa_ref[...], preferred_element_type=jnp.float32) + ba_ref[...]
    g = g_ref[0, 0]
    res = g * o + (1.0 - g) * x_ref[...]
    out_ref[...] = (
        jnp.dot(res, wc_ref[...], preferred_element_type=jnp.float32)
        + bc_ref[...]
    )


def _out_cls(num, den, x, wa, ba, g, wc, bc):
    n = x.shape[0]
    grid = n // BN
    return pl.pallas_call(
        _out_cls_body,
        grid=(grid,),
        in_specs=[
            pl.BlockSpec((BN, HID), lambda i: (i, 0)),
            pl.BlockSpec((BN, 16), lambda i: (i, 0)),
            pl.BlockSpec((BN, HID), lambda i: (i, 0)),
            pl.BlockSpec((HID, HID), lambda i: (0, 0)),
            pl.BlockSpec((1, HID), lambda i: (0, 0)),
            pl.BlockSpec((1, 1), lambda i: (0, 0)),
            pl.BlockSpec((HID, 128), lambda i: (0, 0)),
            pl.BlockSpec((1, 128), lambda i: (0, 0)),
        ],
        out_specs=pl.BlockSpec((BN, 128), lambda i: (i, 0)),
        out_shape=jax.ShapeDtypeStruct((n, 128), jnp.float32),
    )(num, den, x, wa, ba, g, wc, bc)


_SC_MESH = plsc.VectorSubcoreMesh(core_axis_name="c", subcore_axis_name="s")
_SC_PARAMS = pltpu.CompilerParams(use_tc_tiling_on_sc=False)


def _shuf(x, idx):
    return lax.gather(
        x, idx[:, None],
        dimension_numbers=lax.GatherDimensionNumbers(
            offset_dims=(), collapsed_slice_dims=(0,), start_index_map=(0,)),
        slice_sizes=(1,), mode=lax.GatherScatterMode.PROMISE_IN_BOUNDS)


def _zero_shared(msg, shared, s, width):
    zero16 = jnp.zeros((16,), jnp.float32)

    def _zrow(i, carry):
        for j in range(width // 16):
            msg[i, pl.ds(j * 16, 16)] = zero16
        return carry
    lax.fori_loop(0, EC, _zrow, 0)
    zb = s * SUBC_ROWS
    for j in range(SUBC_ROWS // EC):
        pltpu.sync_copy(msg, shared.at[pl.ds(zb + j * EC, EC)])
    rem = SUBC_ROWS % EC
    if rem:
        pltpu.sync_copy(msg.at[pl.ds(0, rem)],
                        shared.at[pl.ds(zb + SUBC_ROWS - rem, rem)])

    @pl.when(s == 0)
    def _():
        pltpu.sync_copy(msg.at[pl.ds(0, 8)], shared.at[pl.ds(SC_HALF, 8)])


def _copy_out(msg, shared, out_hbm, s, base_node):
    r0 = s * SUBC_ROWS
    for j in range(SUBC_ROWS // EC):
        pltpu.sync_copy(shared.at[pl.ds(r0 + j * EC, EC)], msg)
        pltpu.sync_copy(msg, out_hbm.at[pl.ds(base_node + r0 + j * EC, EC)])
    rem = SUBC_ROWS % EC
    if rem:
        off = r0 + SUBC_ROWS - rem
        pltpu.sync_copy(shared.at[pl.ds(off, rem)], msg.at[pl.ds(0, rem)])
        pltpu.sync_copy(msg.at[pl.ds(0, rem)],
                        out_hbm.at[pl.ds(base_node + off, rem)])


def _localize(colb, c):
    base = c * SC_HALF
    for j in range(EC // 16):
        cv = colb[pl.ds(j * 16, 16)]
        loc = cv - base
        ok = (loc >= 0) & (loc < SC_HALF)
        colb[pl.ds(j * 16, 16)] = jnp.where(ok, loc, SC_HALF)


@functools.partial(
    pl.kernel,
    out_type=(jax.ShapeDtypeStruct((E_TOTAL, 16), jnp.float32),
              jax.ShapeDtypeStruct((NPAD, 16), jnp.float32)),
    mesh=_SC_MESH, compiler_params=_SC_PARAMS,
    scratch_types=[pltpu.VMEM((EC,), jnp.int32),
                   pltpu.VMEM((EC,), jnp.int32),
                   pltpu.VMEM((EC, 64), jnp.float32),
                   pltpu.VMEM((EC, 64), jnp.float32),
                   pltpu.VMEM((EC, 16), jnp.float32),
                   pltpu.VMEM_SHARED((SC_HALF + 8, 16), jnp.float32),
                   pltpu.SemaphoreType.DMA])
def _edge_kernel_a(q_hbm, k_hbm, row_hbm, col_hbm, ex_hbm, den_hbm,
                   rowb, colb, qb, kb, exm, shared, sem):
    c = lax.axis_index("c")
    s = lax.axis_index("s")
    lanes = lax.iota(jnp.int32, 16)
    zero16 = jnp.zeros((16,), jnp.float32)
    _zero_shared(exm, shared, s, 16)
    plsc.subcore_barrier()

    def _chunk(t, carry):
        eb = (s + 16 * t) * EC
        pltpu.sync_copy(row_hbm.at[pl.ds(eb, EC)], rowb)
        pltpu.sync_copy(col_hbm.at[pl.ds(eb, EC)], colb)
        cp1 = pltpu.async_copy(k_hbm.at[rowb], kb, sem)
        cp2 = pltpu.async_copy(q_hbm.at[colb], qb, sem)
        cp1.wait()
        cp2.wait()

        def _edge(e, carry2):
            den = zero16
            for h in range(H):
                prod = qb[e, pl.ds(h * 16, 16)] * kb[e, pl.ds(h * 16, 16)]
                for shift in (8, 4, 2, 1):
                    prod = prod + _shuf(prod, lanes ^ shift)
                ex = jnp.exp(prod)
                den = den + jnp.where(lanes == h, ex, 0.0)
            exm[e, pl.ds(0, 16)] = den
            return carry2
        lax.fori_loop(0, EC, _edge, 0)

        @pl.when(c == 0)
        def _():
            pltpu.sync_copy(exm, ex_hbm.at[pl.ds(eb, EC)])
        _localize(colb, c)
        pltpu.sync_copy(exm, shared.at[colb], add=True)
        return carry
    nt = jnp.where(s < NCHT % 16, NCHT // 16 + 1, NCHT // 16)
    lax.fori_loop(0, nt, _chunk, 0)
    plsc.subcore_barrier()
    _copy_out(exm, shared, den_hbm, s, c * SC_HALF)


@functools.partial(
    pl.kernel,
    out_type=jax.ShapeDtypeStruct((NPAD, 64), jnp.float32),
    mesh=_SC_MESH, compiler_params=_SC_PARAMS,
    scratch_types=[pltpu.VMEM((EC,), jnp.int32),
                   pltpu.VMEM((EC,), jnp.int32),
                   pltpu.VMEM((EC, 64), jnp.float32),
                   pltpu.VMEM((EC, 16), jnp.float32),
                   pltpu.VMEM((EC, 64), jnp.float32),
                   pltpu.VMEM_SHARED((SC_HALF + 8, 64), jnp.float32),
                   pltpu.SemaphoreType.DMA])
def _edge_kernel_b(v_hbm, ex_hbm, row_hbm, col_hbm, num_hbm,
                   rowb, colb, vb, exb, msg, shared, sem):
    c = lax.axis_index("c")
    s = lax.axis_index("s")
    _zero_shared(msg, shared, s, 64)
    plsc.subcore_barrier()

    def _chunk(t, carry):
        eb = (s + 16 * t) * EC
        pltpu.sync_copy(row_hbm.at[pl.ds(eb, EC)], rowb)
        pltpu.sync_copy(col_hbm.at[pl.ds(eb, EC)], colb)
        pltpu.sync_copy(ex_hbm.at[pl.ds(eb, EC)], exb)
        pltpu.async_copy(v_hbm.at[rowb], vb, sem).wait()

        def _edge(e, carry2):
            exv = exb[e, pl.ds(0, 16)]
            for h in range(H):
                bex = _shuf(exv, jnp.full((16,), h, jnp.int32))
                msg[e, pl.ds(h * 16, 16)] = vb[e, pl.ds(h * 16, 16)] * bex
            return carry2
        lax.fori_loop(0, EC, _edge, 0)

        _localize(colb, c)
        pltpu.sync_copy(msg, shared.at[colb], add=True)
        return carry
    nt = jnp.where(s < NCHT % 16, NCHT // 16 + 1, NCHT // 16)
    lax.fori_loop(0, nt, _chunk, 0)
    plsc.subcore_barrier()
    _copy_out(msg, shared, num_hbm, s, c * SC_HALF)


def _edge_pass(q_dst, k_src, v_src, row, col):
    ex, den = _edge_kernel_a(q_dst, k_src, row, col)
    num = _edge_kernel_b(v_src, ex, row, col)
    return num, den


def _fold_weights(pn, pe_out):
    wq = pn["Wq"]
    bq = pn["bq"]
    scale = pe_out["p_rel"] / jnp.sqrt(float(D))
    wk = jnp.einsum("ihd,hde->ihe", pn["Wk"].reshape(-1, H, D), pe_out["a_rel"])
    wk = (wk * scale[None, :, None]).reshape(-1, HID)
    bk = jnp.einsum("hd,hde->he", pn["bk"].reshape(H, D), pe_out["a_rel"])
    bk = (bk * scale[:, None]).reshape(HID)
    wv = jnp.einsum("ihd,hde->ihe", pn["Wv"].reshape(-1, H, D), pe_out["m_rel"])
    wv = wv.reshape(-1, HID)
    bv = jnp.einsum("hd,hde->he", pn["bv"].reshape(H, D), pe_out["m_rel"])
    bv = bv.reshape(HID)
    w = jnp.concatenate([wq, wk, wv], axis=1)
    b = jnp.concatenate([bq, bk, bv])[None, :]
    return w, b


def kernel(x_transaction, x_account, params, edge_index_t2a, edge_index_a2t):
    p1, p2 = params["conv1"], params["conv2"]
    e_t2a = "transaction__to__account"
    e_a2t = "account__to__transaction"

    row_t2a, col_t2a = edge_index_t2a[0], edge_index_t2a[1]
    row_a2t, col_a2t = edge_index_a2t[0], edge_index_a2t[1]

    wt, bt = _fold_weights(p1["nodes"]["transaction"], p1["edges"][e_t2a])
    wa_, ba_ = _fold_weights(p1["nodes"]["account"], p1["edges"][e_a2t])
    qt, kt, vt = _proj(x_transaction, wt, bt)
    qa, ka, va = _proj(x_account, wa_, ba_)

    num_a, den_a = _edge_pass(qa, kt, vt, row_t2a, col_t2a)
    num_t, den_t = _edge_pass(qt, ka, va, row_a2t, col_a2t)

    pn = p1["nodes"]["transaction"]
    h_t = _out_stage(num_t, den_t, x_transaction, pn["Wa"], pn["ba"][None, :],
                     jax.nn.sigmoid(pn["skip"]).reshape(1, 1), relu=True)
    pn = p1["nodes"]["account"]
    h_a = _out_stage(num_a, den_a, x_account, pn["Wa"], pn["ba"][None, :],
                     jax.nn.sigmoid(pn["skip"]).reshape(1, 1), relu=True)

    wt2, bt2 = _fold_weights(p2["nodes"]["transaction"], p2["edges"][e_t2a])
    wa2, ba2 = _fold_weights(p2["nodes"]["account"], p2["edges"][e_a2t])
    qt2, _, _ = _proj(h_t, wt2, bt2)
    _, ka2, va2 = _proj(h_a, wa2, ba2)

    num_t2, den_t2 = _edge_pass(qt2, ka2, va2, row_a2t, col_a2t)

    pn = p2["nodes"]["transaction"]
    wc = jnp.zeros((HID, 128), jnp.float32).at[:, :1].set(params["cls_W"])
    bc = jnp.zeros((1, 128), jnp.float32).at[0, 0].set(params["cls_b"][0])
    out = _out_cls(num_t2, den_t2, h_t, pn["Wa"], pn["ba"][None, :],
                   jax.nn.sigmoid(pn["skip"]).reshape(1, 1), wc, bc)
    return out[:, :1]
